# baseline (speedup 1.0000x reference)
import functools

import jax
import jax.numpy as jnp
import numpy as np
from jax import lax
from jax.experimental import pallas as pl
from jax.experimental.pallas import tpu as pltpu
from jax.experimental.pallas import tpu_sc as plsc

EMBED_N = 100000
HIDDEN = 64
FIXED = 26
BATCH = 4096
SEQ = 50
SEQ_PAD = 56

NC, NS, LANES = 2, 16, 16
NW = NC * NS
NB_W = BATCH // NW
NVH = HIDDEN // LANES

TCOLS = 4096


def _g_body(t_ref, g_ref):
    xt = t_ref[...]
    ssq = jnp.sum(xt * xt, axis=0, keepdims=True)
    nrm = jnp.sqrt(ssq)
    scale = jnp.where(nrm > 1.0, 1.0 / (nrm + 1e-7), 1.0)
    rt = jnp.maximum(xt, 0.0) * scale
    gt = rt * rt
    g = gt.T.astype(jnp.bfloat16)
    g_ref[...] = jnp.concatenate([g, g], axis=1)


NBUF = 8

_ACC_PERM = np.concatenate(
    [np.arange(g, g + 32, 2) for g in (0, 0, 32, 32)]
).reshape(4, 16)
_ACC_PERM[1::2] += 1
_ACC_PERM = _ACC_PERM.reshape(-1)


def _sc_body(lab_ref, g_ref, out_ref, idx_v, rows_v, acc_v, sem_g):
    wid = lax.axis_index("c") * NS + lax.axis_index("s")
    b_base = wid * NB_W
    pltpu.sync_copy(lab_ref.at[pl.ds(b_base, NB_W)], idx_v)

    def gather(c):
        pltpu.async_copy(
            g_ref.at[idx_v.at[c, pl.ds(0, SEQ_PAD)]], rows_v.at[c % NBUF], sem_g
        )

    for c0 in range(NBUF - 1):
        gather(c0)

    @pl.loop(0, NB_W)
    def _(c):
        @pl.when(c + NBUF - 1 < NB_W)
        def _():
            gather(c + NBUF - 1)

        p = c % NBUF
        pltpu.make_async_copy(
            g_ref.at[idx_v.at[c, pl.ds(0, SEQ_PAD)]], rows_v.at[p], sem_g
        ).wait()

        accs = [jnp.zeros((LANES,), jnp.float32) for _ in range(NVH)]
        for l in range(SEQ):
            for j in range(NVH // 2):
                xi = rows_v[p, l, pl.ds(LANES * j, LANES)]
                ev = lax.bitcast_convert_type(
                    lax.shift_left(xi, 16), jnp.float32
                )
                od = lax.bitcast_convert_type(
                    lax.bitwise_and(xi, jnp.int32(-65536)), jnp.float32
                )
                accs[2 * j] = accs[2 * j] + ev
                accs[2 * j + 1] = accs[2 * j + 1] + od
        for j in range(NVH):
            acc_v[c, pl.ds(j * LANES, LANES)] = accs[j]

    pltpu.sync_copy(acc_v, out_ref.at[pl.ds(b_base, NB_W)])


_sc_call = functools.partial(
    pl.kernel,
    out_type=jax.ShapeDtypeStruct((BATCH, HIDDEN), jnp.float32),
    mesh=plsc.VectorSubcoreMesh(
        core_axis_name="c", subcore_axis_name="s", num_cores=NC, num_subcores=NS
    ),
    compiler_params=pltpu.CompilerParams(use_tc_tiling_on_sc=False),
    scratch_types=[
        pltpu.VMEM((NB_W, 128), jnp.int32),
        pltpu.VMEM((NBUF, SEQ_PAD, HIDDEN // 2), jnp.int32),
        pltpu.VMEM((NB_W, HIDDEN), jnp.float32),
        pltpu.SemaphoreType.DMA,
    ],
)


def _fin_body(acc_ref, fx_ref, wf_ref, wep_ref, b_ref, o_ref):
    ew = jnp.sqrt(acc_ref[...])
    s = (
        jnp.sum(fx_ref[...] * wf_ref[...], axis=1, keepdims=True)
        + jnp.sum(ew * wep_ref[...], axis=1, keepdims=True)
        + b_ref[0, 0]
    )
    o_ref[...] = jax.nn.sigmoid(s)


@jax.jit
def _impl(label, fixed, table, W, b):
    g2 = pl.pallas_call(
        _g_body,
        grid=(pl.cdiv(EMBED_N, TCOLS),),
        in_specs=[pl.BlockSpec((HIDDEN, TCOLS), lambda i: (0, i))],
        out_specs=pl.BlockSpec((TCOLS, 2 * HIDDEN), lambda i: (i, 0)),
        out_shape=jax.ShapeDtypeStruct((EMBED_N, 2 * HIDDEN), jnp.bfloat16),
    )(table.T)
    g_i = lax.bitcast_convert_type(g2.reshape(EMBED_N, HIDDEN, 2), jnp.int32)
    g = g_i.reshape(2 * EMBED_N, HIDDEN // 2)
    lab_pad = jnp.pad(label.astype(jnp.int32) * 2, ((0, 0), (0, 128 - SEQ)))
    col = jnp.arange(128, dtype=jnp.int32)[None, :]
    row = jnp.arange(BATCH, dtype=jnp.int32)[:, None]
    junk = ((row * 79 + col * 131) % EMBED_N) * 2
    lab_pad = jnp.where(col < SEQ, lab_pad, junk)
    acc = _sc_call(_sc_body)(lab_pad, g)
    wf = W[:, :FIXED]
    we_p = W[:, FIXED:][:, _ACC_PERM]
    out = pl.pallas_call(
        _fin_body,
        out_shape=jax.ShapeDtypeStruct((BATCH, 1), jnp.float32),
    )(acc, fixed, wf, we_p, b.reshape(1, 1))
    return out


def kernel(label, fixed, table, W, b):
    return _impl(label, fixed, table, W, b)


# speedup vs baseline: 5.7928x; 5.7928x over previous
import functools

import jax
import jax.numpy as jnp
from jax import lax
from jax.experimental import pallas as pl
from jax.experimental.pallas import tpu as pltpu
from jax.experimental.pallas import tpu_sc as plsc

EMBED_N = 100000
HIDDEN = 64
FIXED = 26
BATCH = 4096
SEQ = 50
SEQ_PAD = 56

NC, NS, LANES = 2, 16, 16
NW = NC * NS
NB_W = BATCH // NW
NVH = HIDDEN // LANES

TCOLS = 4096
GROWS = pl.cdiv(EMBED_N, TCOLS) * (TCOLS // 2)


def _g_body(t_ref, g_ref):
    xt = t_ref[...]
    ssq = jnp.sum(xt * xt, axis=0, keepdims=True)
    nrm = jnp.sqrt(ssq)
    scale = jnp.where(nrm > 1.0, 1.0 / (nrm + 1e-7), 1.0)
    rt = jnp.maximum(xt, 0.0) * scale
    gt = rt * rt
    ga = gt[:, : TCOLS // 2]
    gb = gt[:, TCOLS // 2 :]
    g_ref[...] = jnp.concatenate([ga.T, gb.T], axis=1)


NBUF = 8


def _sc_body(lab_ref, g_ref, out_ref, idx_v, rows_v, acc_v, sem_g):
    wid = lax.axis_index("c") * NS + lax.axis_index("s")
    b_base = wid * NB_W
    pltpu.sync_copy(lab_ref.at[pl.ds(b_base, NB_W)], idx_v)

    def gather(c):
        pltpu.async_copy(
            g_ref.at[idx_v.at[c, pl.ds(0, SEQ_PAD)]], rows_v.at[c % NBUF], sem_g
        )

    for c0 in range(NBUF - 1):
        gather(c0)

    @pl.loop(0, NB_W)
    def _(c):
        @pl.when(c + NBUF - 1 < NB_W)
        def _():
            gather(c + NBUF - 1)

        p = c % NBUF
        pltpu.make_async_copy(
            g_ref.at[idx_v.at[c, pl.ds(0, SEQ_PAD)]], rows_v.at[p], sem_g
        ).wait()

        accs = [jnp.zeros((LANES,), jnp.float32) for _ in range(NVH)]
        for l in range(SEQ):
            for j in range(NVH):
                accs[j] = accs[j] + rows_v[p, l, pl.ds(j * LANES, LANES)]
        for j in range(NVH):
            acc_v[c, pl.ds(j * LANES, LANES)] = accs[j]

    pltpu.sync_copy(acc_v, out_ref.at[pl.ds(b_base, NB_W)])


_sc_call = functools.partial(
    pl.kernel,
    out_type=jax.ShapeDtypeStruct((BATCH, HIDDEN), jnp.float32),
    mesh=plsc.VectorSubcoreMesh(
        core_axis_name="c", subcore_axis_name="s", num_cores=NC, num_subcores=NS
    ),
    compiler_params=pltpu.CompilerParams(use_tc_tiling_on_sc=False),
    scratch_types=[
        pltpu.VMEM((NB_W, 128), jnp.int32),
        pltpu.VMEM((NBUF, SEQ_PAD, HIDDEN), jnp.float32),
        pltpu.VMEM((NB_W, HIDDEN), jnp.float32),
        pltpu.SemaphoreType.DMA,
    ],
)


def _fin_body(acc_ref, fx_ref, w_ref, b_ref, o_ref):
    ew = jnp.sqrt(acc_ref[...])
    w = w_ref[...]
    wf = w[:, :FIXED]
    we = w[:, FIXED:]
    s = (
        jnp.sum(fx_ref[...] * wf, axis=1, keepdims=True)
        + jnp.sum(ew * we, axis=1, keepdims=True)
        + b_ref[0, 0]
    )
    o_ref[...] = jax.nn.sigmoid(s)


@jax.jit
def _impl(label, fixed, table, W, b):
    g2 = pl.pallas_call(
        _g_body,
        grid=(pl.cdiv(EMBED_N, TCOLS),),
        in_specs=[pl.BlockSpec((HIDDEN, TCOLS), lambda i: (0, i))],
        out_specs=pl.BlockSpec((TCOLS // 2, 2 * HIDDEN), lambda i: (i, 0)),
        out_shape=jax.ShapeDtypeStruct((GROWS, 2 * HIDDEN), jnp.float32),
    )(table.T)
    g = g2.reshape(2 * GROWS, HIDDEN)
    lab_pad = jnp.pad(label.astype(jnp.int32), ((0, 0), (0, 128 - SEQ)))
    col = jnp.arange(128, dtype=jnp.int32)[None, :]
    row = jnp.arange(BATCH, dtype=jnp.int32)[:, None]
    junk = (row * 79 + col * 131) % EMBED_N
    lab_pad = jnp.where(col < SEQ, lab_pad, junk)
    lab_pad = (
        (lab_pad & -TCOLS)
        | ((lab_pad & (TCOLS // 2 - 1)) << 1)
        | ((lab_pad >> 11) & 1)
    )
    acc = _sc_call(_sc_body)(lab_pad, g)
    out = pl.pallas_call(
        _fin_body,
        out_shape=jax.ShapeDtypeStruct((BATCH, 1), jnp.float32),
    )(acc, fixed, W, b.reshape(1, 1))
    return out


def kernel(label, fixed, table, W, b):
    return _impl(label, fixed, table, W, b)
